# baseline (speedup 1.0000x reference)
import jax
import jax.numpy as jnp
from jax import lax
from jax.experimental import pallas as pl
from jax.experimental.pallas import tpu as pltpu
from jax.experimental.pallas import tpu_sc as plsc

_K = 64
_M = 128
_N = 32768
_NC, _NS, _L = 2, 16, 16
_NW = _NC * _NS
_RPW = _M // _NW
_NV = _N // _L
_MIN32 = -2147483648


def _f2k(v):
    b = lax.bitcast_convert_type(v, jnp.int32)
    return b ^ (lax.shift_right_arithmetic(b, 31) & jnp.int32(0x7FFFFFFF))


def _k2f(kvec):
    b = jnp.where(kvec >= 0, kvec, kvec ^ jnp.int32(0x7FFFFFFF))
    return lax.bitcast_convert_type(b, jnp.float32)


def _process_row(x_v, buf_v):
    neg_inf = jnp.full((_L,), -jnp.inf, jnp.float32)

    def amax(j, accs):
        base = j * 8 * _L
        return tuple(
            jnp.maximum(a, x_v[pl.ds(base + g * _L, _L)])
            for g, a in enumerate(accs)
        )

    accs = lax.fori_loop(0, _NV // 8, amax, (neg_inf,) * 8)
    kaccs = [_f2k(a) for a in accs]

    def bstep(i, ub):
        bit = jnp.int32(31) - i
        cand = ub | lax.shift_left(jnp.int32(1), bit)
        cs = cand ^ _MIN32
        cntv = jnp.zeros((_L,), jnp.int32)
        for ka in kaccs:
            cntv = cntv + (ka >= cs).astype(jnp.int32)
        cnt = jnp.sum(cntv)
        return jnp.where(cnt >= _K, cand, ub)

    ub = lax.fori_loop(0, 32, bstep, jnp.int32(0))
    m2f_v = _k2f(jnp.full((_L,), ub ^ _MIN32, jnp.int32))

    lane = lax.iota(jnp.int32, _L)

    def cstep(i, off_v):
        base = i * 4 * _L
        for u in range(4):
            v = x_v[pl.ds(base + u * _L, _L)]
            msk = v >= m2f_v
            plsc.store_scatter(buf_v, [off_v * _L + lane], v, mask=msk)
            off_v = off_v + msk.astype(jnp.int32)
        return off_v

    cnt_v = lax.fori_loop(0, _NV // 4, cstep, jnp.zeros((_L,), jnp.int32))
    mx = plsc.cummax(cnt_v)[15]
    mc4 = (mx + jnp.int32(3)) & jnp.int32(~3)

    def clean(i, carry):
        iv = jnp.full((_L,), i, jnp.int32)
        sl = pl.ds(i * _L, _L)
        buf_v[sl] = jnp.where(iv < cnt_v, buf_v[sl], neg_inf)
        return carry

    lax.fori_loop(0, mc4, clean, jnp.int32(0))

    def bstep2(i, ub2):
        bit = jnp.int32(31) - i
        cand = ub2 | lax.shift_left(jnp.int32(1), bit)
        cs = cand ^ _MIN32

        def inner(j, acc):
            base = j * 4 * _L
            for u in range(4):
                kv = _f2k(buf_v[pl.ds(base + u * _L, _L)])
                acc = acc + (kv >= cs).astype(jnp.int32)
            return acc

        cntv = lax.fori_loop(0, mc4 // 4, inner, jnp.zeros((_L,), jnp.int32))
        cnt = jnp.sum(cntv)
        return jnp.where(cnt >= _K, cand, ub2)

    ub2 = lax.fori_loop(0, 32, bstep2, jnp.int32(0))
    t_v = _k2f(jnp.full((_L,), ub2 ^ _MIN32, jnp.int32))

    zero = jnp.zeros((_L,), jnp.float32)

    def mstep(i, carry):
        base = i * 4 * _L
        for u in range(4):
            sl = pl.ds(base + u * _L, _L)
            v = x_v[sl]
            x_v[sl] = jnp.where(v >= t_v, v, zero)
        return carry

    lax.fori_loop(0, _NV // 4, mstep, jnp.int32(0))


def _sc_body(in_hbm, out_hbm, xa_v, xb_v, buf_v,
             sin_a, sin_b, sout_a, sout_b):
    wid = lax.axis_index("s") * _NC + lax.axis_index("c")
    bufs = (xa_v, xb_v)
    sins = (sin_a, sin_b)
    souts = (sout_a, sout_b)
    in_h = {}
    out_h = [None, None]

    in_h[0] = pltpu.async_copy(in_hbm.at[wid * _RPW], bufs[0], sins[0])
    for rr in range(_RPW):
        b = rr % 2
        if rr + 1 < _RPW:
            nb = (rr + 1) % 2
            if out_h[nb] is not None:
                out_h[nb].wait()
            in_h[rr + 1] = pltpu.async_copy(
                in_hbm.at[wid * _RPW + rr + 1], bufs[nb], sins[nb])
        in_h[rr].wait()
        _process_row(bufs[b], buf_v)
        out_h[b] = pltpu.async_copy(bufs[b], out_hbm.at[wid * _RPW + rr],
                                    souts[b])
    for b in range(2):
        if out_h[b] is not None:
            out_h[b].wait()


@jax.jit
def kernel(input):
    mesh = plsc.VectorSubcoreMesh(
        core_axis_name="c", subcore_axis_name="s",
        num_cores=_NC, num_subcores=_NS,
    )
    run = pl.kernel(
        _sc_body,
        out_type=jax.ShapeDtypeStruct((_M, _N), jnp.float32),
        mesh=mesh,
        compiler_params=pltpu.CompilerParams(needs_layout_passes=False),
        scratch_types=[
            pltpu.VMEM((_N,), jnp.float32),
            pltpu.VMEM((_N,), jnp.float32),
            pltpu.VMEM((_N + 4 * _L,), jnp.float32),
            pltpu.SemaphoreType.DMA,
            pltpu.SemaphoreType.DMA,
            pltpu.SemaphoreType.DMA,
            pltpu.SemaphoreType.DMA,
        ],
    )
    return run(input)


# speedup vs baseline: 1.0688x; 1.0688x over previous
import jax
import jax.numpy as jnp
from jax import lax
from jax.experimental import pallas as pl
from jax.experimental.pallas import tpu as pltpu
from jax.experimental.pallas import tpu_sc as plsc

_K = 64
_M = 128
_N = 32768
_NC, _NS, _L = 2, 16, 16
_NW = _NC * _NS
_RPW = _M // _NW
_NV = _N // _L
_MIN32 = -2147483648


def _f2k(v):
    b = lax.bitcast_convert_type(v, jnp.int32)
    return b ^ (lax.shift_right_arithmetic(b, 31) & jnp.int32(0x7FFFFFFF))


def _k2f(kvec):
    b = jnp.where(kvec >= 0, kvec, kvec ^ jnp.int32(0x7FFFFFFF))
    return lax.bitcast_convert_type(b, jnp.float32)


def _process_row(x_v, buf_v):
    neg_inf = jnp.full((_L,), -jnp.inf, jnp.float32)

    def amax(j, accs):
        base = j * 8 * _L
        return tuple(
            jnp.maximum(a, x_v[pl.ds(base + g * _L, _L)])
            for g, a in enumerate(accs)
        )

    accs = lax.fori_loop(0, _NV // 8, amax, (neg_inf,) * 8)
    kaccs = [_f2k(a) for a in accs]

    def bstep(i, ub):
        bit = jnp.int32(31) - i
        cand = ub | lax.shift_left(jnp.int32(1), bit)
        cs = cand ^ _MIN32
        cntv = jnp.zeros((_L,), jnp.int32)
        for ka in kaccs:
            cntv = cntv + (ka >= cs).astype(jnp.int32)
        cnt = jnp.sum(cntv)
        return jnp.where(cnt >= _K, cand, ub)

    ub = lax.fori_loop(0, 32, bstep, jnp.int32(0))
    m2f_v = _k2f(jnp.full((_L,), ub ^ _MIN32, jnp.int32))

    lane = lax.iota(jnp.int32, _L)

    def cstep(i, addr_v):
        base = i * 8 * _L
        for u in range(8):
            v = x_v[pl.ds(base + u * _L, _L)]
            msk = v >= m2f_v
            plsc.store_scatter(buf_v, [addr_v], v, mask=msk)
            addr_v = addr_v + lax.shift_left(msk.astype(jnp.int32), 4)
        return addr_v

    addr_v = lax.fori_loop(0, _NV // 8, cstep, lane)
    cnt_v = lax.shift_right_logical(addr_v - lane, 4)
    mx = plsc.cummax(cnt_v)[15]
    mc4 = (mx + jnp.int32(3)) & jnp.int32(~3)

    ninf_key = _f2k(neg_inf)

    def clean(i, carry):
        iv = jnp.full((_L,), i, jnp.int32)
        sl = pl.ds(i * _L, _L)
        kv = jnp.where(iv < cnt_v, _f2k(buf_v[sl]), ninf_key)
        buf_v[sl] = lax.bitcast_convert_type(kv, jnp.float32)
        return carry

    lax.fori_loop(0, mc4, clean, jnp.int32(0))

    def bstep2(i, ub2):
        bit = jnp.int32(31) - i
        cand = ub2 | lax.shift_left(jnp.int32(1), bit)
        cs = cand ^ _MIN32

        def inner(j, acc):
            base = j * 4 * _L
            for u in range(4):
                kv = lax.bitcast_convert_type(
                    buf_v[pl.ds(base + u * _L, _L)], jnp.int32)
                acc = acc + (kv >= cs).astype(jnp.int32)
            return acc

        cntv = lax.fori_loop(0, mc4 // 4, inner, jnp.zeros((_L,), jnp.int32))
        cnt = jnp.sum(cntv)
        return jnp.where(cnt >= _K, cand, ub2)

    ub2 = lax.fori_loop(0, 32, bstep2, jnp.int32(0))
    t_v = _k2f(jnp.full((_L,), ub2 ^ _MIN32, jnp.int32))

    zero = jnp.zeros((_L,), jnp.float32)

    def mstep(i, carry):
        base = i * 8 * _L
        for u in range(8):
            sl = pl.ds(base + u * _L, _L)
            v = x_v[sl]
            x_v[sl] = jnp.where(v >= t_v, v, zero)
        return carry

    lax.fori_loop(0, _NV // 8, mstep, jnp.int32(0))


def _sc_body(in_hbm, out_hbm, xa_v, xb_v, buf_v,
             sin_a, sin_b, sout_a, sout_b):
    wid = lax.axis_index("s") * _NC + lax.axis_index("c")
    bufs = (xa_v, xb_v)
    sins = (sin_a, sin_b)
    souts = (sout_a, sout_b)
    in_h = {}
    out_h = [None, None]

    in_h[0] = pltpu.async_copy(in_hbm.at[wid * _RPW], bufs[0], sins[0])
    for rr in range(_RPW):
        b = rr % 2
        if rr + 1 < _RPW:
            nb = (rr + 1) % 2
            if out_h[nb] is not None:
                out_h[nb].wait()
            in_h[rr + 1] = pltpu.async_copy(
                in_hbm.at[wid * _RPW + rr + 1], bufs[nb], sins[nb])
        in_h[rr].wait()
        _process_row(bufs[b], buf_v)
        out_h[b] = pltpu.async_copy(bufs[b], out_hbm.at[wid * _RPW + rr],
                                    souts[b])
    for b in range(2):
        if out_h[b] is not None:
            out_h[b].wait()


@jax.jit
def kernel(input):
    mesh = plsc.VectorSubcoreMesh(
        core_axis_name="c", subcore_axis_name="s",
        num_cores=_NC, num_subcores=_NS,
    )
    run = pl.kernel(
        _sc_body,
        out_type=jax.ShapeDtypeStruct((_M, _N), jnp.float32),
        mesh=mesh,
        compiler_params=pltpu.CompilerParams(needs_layout_passes=False),
        scratch_types=[
            pltpu.VMEM((_N,), jnp.float32),
            pltpu.VMEM((_N,), jnp.float32),
            pltpu.VMEM((_N + 4 * _L,), jnp.float32),
            pltpu.SemaphoreType.DMA,
            pltpu.SemaphoreType.DMA,
            pltpu.SemaphoreType.DMA,
            pltpu.SemaphoreType.DMA,
        ],
    )
    return run(input)
